# Initial kernel scaffold; baseline (speedup 1.0000x reference)
import jax
import jax.numpy as jnp
from jax.experimental import pallas as pl


def kernel(x, edge_index, batch, W1, b1, W2, b2, W3, b3, Wc, bc):
    raise NotImplementedError("write your pallas kernel here")


# speedup vs baseline: 1.1110x; 1.1110x over previous
import jax
import jax.numpy as jnp
from jax.experimental import pallas as pl

EPS = 1e-6
C_IN = 1.0
C_OUT = 1.0
C_LIN = 4.0
GIN_EPS = 0.0
N_GRAPHS = 64
TILE = 1000


def _expz(u, c):
    sc = c ** 0.5
    s = jnp.sum(u * u, axis=-1, keepdims=True)
    lnorm = jnp.sqrt(jnp.clip(s + EPS, 1e-6, None))
    cut = jnp.minimum(lnorm, 50.0)
    a = cut / sc
    sinh_a = 0.5 * (jnp.exp(a) - jnp.exp(-a))
    tail = sc * sinh_a * u / lnorm
    nrm = jnp.sqrt(jnp.clip(jnp.sum(tail * tail, axis=-1, keepdims=True), 1e-12, None))
    tail = tail * jnp.minimum(1.0, 1000.0 / nrm)
    head = jnp.sqrt(c + jnp.sum(tail * tail, axis=-1, keepdims=True))
    return head, tail


def _logz(head, tail, c):
    sc = c ** 0.5
    z = jnp.clip(head / sc + EPS, 1.0 + EPS, None)
    dist = sc * jnp.log(z + jnp.sqrt(z * z - 1.0))
    tn = jnp.sqrt(jnp.clip(jnp.sum(tail * tail, axis=-1, keepdims=True) + EPS, 1e-12, None))
    return dist * tail / tn


def _k0(x_ref, o_ref):
    h, t = _expz(x_ref[...], C_IN)
    o_ref[...] = _logz(h, t, C_IN)


def _block_math(ht, agg, w, b):
    out = (1.0 + GIN_EPS) * ht + agg
    h1h, h1t = _expz(out, C_IN)
    t = _logz(h1h, h1t, C_LIN)
    mx = jax.lax.dot_general(t, w, (((1,), (1,)), ((), ())),
                             precision=jax.lax.Precision.HIGHEST,
                             preferred_element_type=jnp.float32) + b
    h2h, h2t = _expz(mx, C_LIN)
    xt = jax.nn.relu(_logz(h2h, h2t, C_LIN))
    h3h, h3t = _expz(xt, C_OUT)
    return _logz(h3h, h3t, C_OUT)


def _kblock(ht_ref, agg_ref, w_ref, b_ref, o_ref):
    o_ref[...] = _block_math(ht_ref[...], agg_ref[...], w_ref[...], b_ref[...])


def _kblock_last(ht_ref, agg_ref, w_ref, b_ref, wc_ref, o_ref, y_ref):
    ht = _block_math(ht_ref[...], agg_ref[...], w_ref[...], b_ref[...])
    o_ref[...] = ht
    y_ref[...] = jax.lax.dot_general(ht, wc_ref[...], (((1,), (1,)), ((), ())),
                                     precision=jax.lax.Precision.HIGHEST,
                                     preferred_element_type=jnp.float32)


def _call_k0(x):
    n, d = x.shape
    return pl.pallas_call(
        _k0,
        grid=(n // TILE,),
        in_specs=[pl.BlockSpec((TILE, d), lambda i: (i, 0))],
        out_specs=pl.BlockSpec((TILE, d), lambda i: (i, 0)),
        out_shape=jax.ShapeDtypeStruct((n, d), x.dtype),
    )(x)


def _call_block(ht, agg, w, b):
    n, din = ht.shape
    dout = w.shape[0]
    return pl.pallas_call(
        _kblock,
        grid=(n // TILE,),
        in_specs=[
            pl.BlockSpec((TILE, din), lambda i: (i, 0)),
            pl.BlockSpec((TILE, din), lambda i: (i, 0)),
            pl.BlockSpec((dout, din), lambda i: (0, 0)),
            pl.BlockSpec((1, dout), lambda i: (0, 0)),
        ],
        out_specs=pl.BlockSpec((TILE, dout), lambda i: (i, 0)),
        out_shape=jax.ShapeDtypeStruct((n, dout), ht.dtype),
    )(ht, agg, w, b)


def _call_block_last(ht, agg, w, b, wc):
    n, din = ht.shape
    dout = w.shape[0]
    return pl.pallas_call(
        _kblock_last,
        grid=(n // TILE,),
        in_specs=[
            pl.BlockSpec((TILE, din), lambda i: (i, 0)),
            pl.BlockSpec((TILE, din), lambda i: (i, 0)),
            pl.BlockSpec((dout, din), lambda i: (0, 0)),
            pl.BlockSpec((1, dout), lambda i: (0, 0)),
            pl.BlockSpec((1, dout), lambda i: (0, 0)),
        ],
        out_specs=[
            pl.BlockSpec((TILE, dout), lambda i: (i, 0)),
            pl.BlockSpec((TILE, 1), lambda i: (i, 0)),
        ],
        out_shape=[
            jax.ShapeDtypeStruct((n, dout), ht.dtype),
            jax.ShapeDtypeStruct((n, 1), ht.dtype),
        ],
    )(ht, agg, w, b, wc)


def kernel(x, edge_index, batch, W1, b1, W2, b2, W3, b3, Wc, bc):
    n = x.shape[0]
    src = edge_index[0]
    dst = edge_index[1]
    ht = _call_k0(x)
    for w, b in ((W1, b1), (W2, b2)):
        agg = jax.ops.segment_sum(ht[src], dst, num_segments=n)
        ht = _call_block(ht, agg, w, b.reshape(1, -1))
    agg = jax.ops.segment_sum(ht[src], dst, num_segments=n)
    _, y = _call_block_last(ht, agg, W3, b3.reshape(1, -1), Wc[:, 1:])
    pooled = jax.ops.segment_sum(y, batch, num_segments=N_GRAPHS)
    return pooled + bc
